# baseline (speedup 1.0000x reference)
import functools

import jax
import jax.numpy as jnp
from jax import lax
from jax.experimental import pallas as pl
from jax.experimental.pallas import tpu as pltpu
from jax.experimental.pallas import tpu_sc as plsc

N_STRUCT = 512
LANES = 16
K = 128


def _lower_bound(gather_fn, targets, n, steps):
    lo = jnp.zeros((LANES,), jnp.int32)
    hi = jnp.full((LANES,), n, jnp.int32)
    for _ in range(steps):
        active = lo < hi
        mid = jnp.right_shift(lo + hi, 1)
        midc = jnp.minimum(mid, n - 1)
        vals = gather_fn(midc)
        pred = vals < targets
        lo = jnp.where(active & pred, mid + 1, lo)
        hi = jnp.where(active & (~pred), mid, hi)
    return lo


def _steps_for(n):
    s = 1
    while (1 << s) < n:
        s += 1
    return s + 1


def _make_body(n, n_rows, n_sample, hidden):
    coarse_steps = _steps_for(n_sample)
    fine_steps = _steps_for(K)

    def body(batch_hbm, sample_hbm, b1_hbm, w2_hbm, b2_hbm, out_hbm,
             sample_v, b1_v, w2_v, b2_v, rows_lo_v, rows_up_v,
             counts_v, v_v, out_v, sem_lo, sem_up):
        wid = lax.axis_index("s") * 2 + lax.axis_index("c")
        pltpu.sync_copy(sample_hbm, sample_v)
        pltpu.sync_copy(b1_hbm, b1_v)
        pltpu.sync_copy(w2_hbm, w2_v)
        pltpu.sync_copy(b2_hbm, b2_v)
        iota = lax.iota(jnp.int32, LANES)

        t_lo = wid * LANES + iota
        t_up = t_lo + 1

        def coarse(idx):
            return plsc.load_gather(sample_v, [idx])

        s_lo = _lower_bound(coarse, t_lo, n_sample, coarse_steps)
        s_up = _lower_bound(coarse, t_up, n_sample, coarse_steps)

        r_lo = jnp.clip(s_lo - 1, 0, n_rows - 1)
        r_up = jnp.clip(s_up - 1, 0, n_rows - 1)
        w_lo = jnp.minimum(r_lo * K, n - K)
        w_up = jnp.minimum(r_up * K, n - K)
        cps = []
        for l in range(LANES):
            o_lo = pl.multiple_of(w_lo[l], 8)
            o_up = pl.multiple_of(w_up[l], 8)
            cps.append(pltpu.async_copy(
                batch_hbm.at[pl.ds(o_lo, K)], rows_lo_v.at[l], sem_lo))
            cps.append(pltpu.async_copy(
                batch_hbm.at[pl.ds(o_up, K)], rows_up_v.at[l], sem_up))

        accs = [jnp.zeros((LANES,), jnp.float32) for _ in range(6)]
        for c in range(hidden // LANES):
            x = b1_v[pl.ds(c * LANES, LANES)]
            s = x / (1.0 + jnp.exp(-x))
            ivec = c * LANES + iota
            for j in range(6):
                w = plsc.load_gather(w2_v, [ivec, jnp.full((LANES,), j,
                                                           jnp.int32)])
                accs[j] = accs[j] + s * w
        v = b2_v[...]
        for j in range(6):
            v = jnp.where(iota == j, v + jnp.sum(accs[j]), v)
        v_v[...] = v

        for cp in cps:
            cp.wait()

        def fine(rows_v, targets, s_idx, w):
            def g(off):
                return plsc.load_gather(rows_v, [iota, off])
            off = _lower_bound(g, targets, K, fine_steps)
            return jnp.where(s_idx == 0, 0, w + off)

        pos_lo = fine(rows_lo_v, t_lo, s_lo, w_lo)
        pos_up = fine(rows_up_v, t_up, s_up, w_up)
        counts_v[...] = (pos_up - pos_lo).astype(jnp.float32)

        for k in range(6):
            p = k * LANES + iota
            b_local = p // 6
            j = p - 6 * b_local
            cnt = plsc.load_gather(counts_v, [b_local])
            vv = plsc.load_gather(v_v, [j])
            out_v[pl.ds(k * LANES, LANES)] = cnt * vv
        pltpu.sync_copy(out_v, out_hbm.at[pl.ds(wid * LANES * 6, LANES * 6)])

    return body


def kernel(pos, batch, atomic_numbers, W1, b1, W2, b2):
    n = pos.shape[0]
    hidden = b1.shape[0]

    batch_i32 = batch.astype(jnp.int32)
    n_rows = -(-n // K)
    n_sample = -(-(n_rows + 5) // 16) * 16
    sample = jnp.concatenate(
        [batch_i32[::K],
         jnp.full((n_sample - n_rows,), N_STRUCT, jnp.int32)])
    b2_pad = jnp.zeros((LANES,), jnp.float32).at[:6].set(
        b2.astype(jnp.float32))

    mesh = plsc.VectorSubcoreMesh(core_axis_name="c", subcore_axis_name="s")
    run = functools.partial(
        pl.kernel,
        mesh=mesh,
        compiler_params=pltpu.CompilerParams(needs_layout_passes=False),
        out_type=jax.ShapeDtypeStruct((N_STRUCT * 6,), jnp.float32),
        scratch_types=[
            pltpu.VMEM((n_sample,), jnp.int32),
            pltpu.VMEM((hidden,), jnp.float32),
            pltpu.VMEM((hidden, 6), jnp.float32),
            pltpu.VMEM((LANES,), jnp.float32),
            pltpu.VMEM((LANES, K), jnp.int32),
            pltpu.VMEM((LANES, K), jnp.int32),
            pltpu.VMEM((LANES,), jnp.float32),
            pltpu.VMEM((LANES,), jnp.float32),
            pltpu.VMEM((LANES * 6,), jnp.float32),
            pltpu.SemaphoreType.DMA,
            pltpu.SemaphoreType.DMA,
        ],
    )(_make_body(n, n_rows, n_sample, hidden))

    forces = jnp.zeros((n, 3), jnp.float32)
    energy = jnp.zeros((N_STRUCT,), jnp.float32)
    stress = run(batch_i32, sample, b1.astype(jnp.float32),
                 W2.astype(jnp.float32), b2_pad).reshape(N_STRUCT, 6)
    return (forces, energy, stress)


# speedup vs baseline: 1.1131x; 1.1131x over previous
import functools

import jax
import jax.numpy as jnp
from jax import lax
from jax.experimental import pallas as pl
from jax.experimental.pallas import tpu as pltpu
from jax.experimental.pallas import tpu_sc as plsc

N_STRUCT = 512
LANES = 16
K = 128
NS = 16


def _lower_bound(gather_fn, targets, n, steps):
    lo = jnp.zeros((LANES,), jnp.int32)
    hi = jnp.full((LANES,), n, jnp.int32)
    for _ in range(steps):
        active = lo < hi
        mid = jnp.right_shift(lo + hi, 1)
        midc = jnp.minimum(mid, n - 1)
        vals = gather_fn(midc)
        pred = vals < targets
        lo = jnp.where(active & pred, mid + 1, lo)
        hi = jnp.where(active & (~pred), mid, hi)
    return lo


def _steps_for(n):
    s = 1
    while (1 << s) < n:
        s += 1
    return s + 1


def _make_body(n, n_rows, hidden):
    rpt = -(-n_rows // NS)
    n_virt = rpt * NS
    rpt_vecs = -(-rpt // LANES)
    slab_words = rpt * K
    last_len = n - (NS - 1) * slab_words
    coarse_steps = _steps_for(n_virt)
    fine_steps = _steps_for(K)
    w2_off = hidden
    b2_off = hidden + hidden * 6

    def body(batch_hbm, params_hbm, out_hbm,
             slab_v, stage_v, sample_v, params_v, rows_lo_v, rows_up_v,
             counts_v, v_v, out_v, shared_smp, sem_lo, sem_up):
        cid = lax.axis_index("c")
        sid = lax.axis_index("s")
        wid = sid * 2 + cid
        iota = lax.iota(jnp.int32, LANES)

        start = pl.multiple_of(sid * slab_words, 8)

        @pl.when(sid < NS - 1)
        def _():
            pltpu.sync_copy(batch_hbm.at[pl.ds(start, slab_words)], slab_v)

        @pl.when(sid == NS - 1)
        def _():
            pltpu.sync_copy(
                batch_hbm.at[pl.ds(pl.multiple_of((NS - 1) * slab_words, 8),
                                   last_len)],
                slab_v.at[pl.ds(0, last_len)])

        for c in range(rpt_vecs):
            i = c * LANES + iota
            g_row = sid * rpt + i
            idx = jnp.minimum(i * K, (rpt - 1) * K)
            vals = plsc.load_gather(slab_v, [idx])
            ok = (i < rpt) & (g_row < n_rows)
            stage_v[pl.ds(c * LANES, LANES)] = jnp.where(
                ok, vals, jnp.int32(N_STRUCT))
        stage_w = rpt_vecs * LANES
        pltpu.sync_copy(stage_v,
                        shared_smp.at[pl.ds(pl.multiple_of(sid * stage_w, 8),
                                            stage_w)])
        plsc.subcore_barrier()
        pltpu.sync_copy(shared_smp, sample_v)
        pltpu.sync_copy(params_hbm, params_v)

        t_lo = wid * LANES + iota
        t_up = t_lo + 1

        def coarse(idx):
            row = idx // rpt
            col = idx - row * rpt
            return plsc.load_gather(sample_v, [row * stage_w + col])

        s_lo = _lower_bound(coarse, t_lo, n_virt, coarse_steps)
        s_up = _lower_bound(coarse, t_up, n_virt, coarse_steps)

        r_lo = jnp.clip(s_lo - 1, 0, n_rows - 1)
        r_up = jnp.clip(s_up - 1, 0, n_rows - 1)
        w_lo = jnp.minimum(r_lo * K, n - K)
        w_up = jnp.minimum(r_up * K, n - K)
        cps = []
        for l in range(LANES):
            o_lo = pl.multiple_of(w_lo[l], 8)
            o_up = pl.multiple_of(w_up[l], 8)
            cps.append(pltpu.async_copy(
                batch_hbm.at[pl.ds(o_lo, K)], rows_lo_v.at[l], sem_lo))
            cps.append(pltpu.async_copy(
                batch_hbm.at[pl.ds(o_up, K)], rows_up_v.at[l], sem_up))

        accs = [jnp.zeros((LANES,), jnp.float32) for _ in range(6)]
        for c in range(hidden // LANES):
            x = params_v[pl.ds(c * LANES, LANES)]
            s = x / (1.0 + jnp.exp(-x))
            row = w2_off + (c * LANES + iota) * 6
            for j in range(6):
                w = plsc.load_gather(params_v, [row + j])
                accs[j] = accs[j] + s * w
        b2g = plsc.load_gather(params_v, [b2_off + jnp.minimum(iota, 5)])
        v = jnp.where(iota < 6, b2g, 0.0)
        for j in range(6):
            v = jnp.where(iota == j, v + jnp.sum(accs[j]), v)
        v_v[...] = v

        for cp in cps:
            cp.wait()

        def fine(rows_v, targets, s_idx, w):
            def g(off):
                return plsc.load_gather(rows_v, [iota, off])
            off = _lower_bound(g, targets, K, fine_steps)
            return jnp.where(s_idx == 0, 0, w + off)

        pos_lo = fine(rows_lo_v, t_lo, s_lo, w_lo)
        pos_up = fine(rows_up_v, t_up, s_up, w_up)
        counts_v[...] = (pos_up - pos_lo).astype(jnp.float32)

        for k in range(6):
            p = k * LANES + iota
            b_local = p // 6
            j = p - 6 * b_local
            cnt = plsc.load_gather(counts_v, [b_local])
            vv = plsc.load_gather(v_v, [j])
            out_v[pl.ds(k * LANES, LANES)] = cnt * vv
        pltpu.sync_copy(out_v, out_hbm.at[pl.ds(wid * LANES * 6, LANES * 6)])

    return body


def kernel(pos, batch, atomic_numbers, W1, b1, W2, b2):
    n = pos.shape[0]
    hidden = b1.shape[0]

    batch_i32 = batch.astype(jnp.int32)
    n_rows = -(-n // K)
    rpt = -(-n_rows // NS)
    rpt_vecs = -(-rpt // LANES)

    p_len = hidden + hidden * 6 + 6
    p_pad = -(-p_len // 16) * 16
    params = jnp.concatenate(
        [b1.astype(jnp.float32),
         jnp.reshape(W2.astype(jnp.float32), (-1,)),
         b2.astype(jnp.float32),
         jnp.zeros((p_pad - p_len,), jnp.float32)])

    mesh = plsc.VectorSubcoreMesh(core_axis_name="c", subcore_axis_name="s")
    run = functools.partial(
        pl.kernel,
        mesh=mesh,
        compiler_params=pltpu.CompilerParams(needs_layout_passes=False),
        out_type=jax.ShapeDtypeStruct((N_STRUCT * 6,), jnp.float32),
        scratch_types=[
            pltpu.VMEM((rpt * K,), jnp.int32),
            pltpu.VMEM((rpt_vecs * LANES,), jnp.int32),
            pltpu.VMEM((NS * rpt_vecs * LANES,), jnp.int32),
            pltpu.VMEM((p_pad,), jnp.float32),
            pltpu.VMEM((LANES, K), jnp.int32),
            pltpu.VMEM((LANES, K), jnp.int32),
            pltpu.VMEM((LANES,), jnp.float32),
            pltpu.VMEM((LANES,), jnp.float32),
            pltpu.VMEM((LANES * 6,), jnp.float32),
            pltpu.VMEM_SHARED((NS * rpt_vecs * LANES,), jnp.int32),
            pltpu.SemaphoreType.DMA,
            pltpu.SemaphoreType.DMA,
        ],
    )(_make_body(n, n_rows, hidden))

    forces = jnp.zeros((n, 3), jnp.float32)
    energy = jnp.zeros((N_STRUCT,), jnp.float32)
    stress = run(batch_i32, params).reshape(N_STRUCT, 6)
    return (forces, energy, stress)


# speedup vs baseline: 1.1652x; 1.0468x over previous
import functools

import jax
import jax.numpy as jnp
from jax import lax
from jax.experimental import pallas as pl
from jax.experimental.pallas import tpu as pltpu
from jax.experimental.pallas import tpu_sc as plsc

N_STRUCT = 512
LANES = 16
K = 128


def _lower_bound(gather_fn, targets, n, steps):
    lo = jnp.zeros((LANES,), jnp.int32)
    hi = jnp.full((LANES,), n, jnp.int32)
    for _ in range(steps):
        active = lo < hi
        mid = jnp.right_shift(lo + hi, 1)
        midc = jnp.minimum(mid, n - 1)
        vals = gather_fn(midc)
        pred = vals < targets
        lo = jnp.where(active & pred, mid + 1, lo)
        hi = jnp.where(active & (~pred), mid, hi)
    return lo


def _steps_for(n):
    s = 1
    while (1 << s) < n:
        s += 1
    return s + 1


def _make_body(n, n_rows, n_sample, hidden):
    coarse_steps = _steps_for(n_sample)
    fine_steps = _steps_for(K)
    p_off = n_sample
    w2_off = p_off + hidden
    b2_off = w2_off + hidden * 6

    def f32_gather(ref, idx):
        return plsc.bitcast(plsc.load_gather(ref, [idx]), jnp.float32)

    def body(batch_hbm, aux_hbm, out_hbm,
             aux_v, rows_lo_v, rows_up_v,
             counts_v, v_v, out_v, sem_lo, sem_up, sem_p):
        wid = lax.axis_index("s") * 2 + lax.axis_index("c")
        p_len = b2_off + LANES - p_off
        cp_params = pltpu.async_copy(
            aux_hbm.at[pl.ds(p_off, p_len)],
            aux_v.at[pl.ds(p_off, p_len)], sem_p)
        pltpu.sync_copy(aux_hbm.at[pl.ds(0, n_sample)],
                        aux_v.at[pl.ds(0, n_sample)])
        iota = lax.iota(jnp.int32, LANES)

        t_lo = wid * LANES + iota
        t_up = t_lo + 1

        def coarse(idx):
            return plsc.load_gather(aux_v, [idx])

        s_lo = _lower_bound(coarse, t_lo, n_sample, coarse_steps)
        s_up = _lower_bound(coarse, t_up, n_sample, coarse_steps)

        r_lo = jnp.clip(s_lo - 1, 0, n_rows - 1)
        r_up = jnp.clip(s_up - 1, 0, n_rows - 1)
        w_lo = jnp.minimum(r_lo * K, n - K)
        w_up = jnp.minimum(r_up * K, n - K)
        cps = []
        for l in range(LANES):
            o_lo = pl.multiple_of(w_lo[l], 8)
            o_up = pl.multiple_of(w_up[l], 8)
            cps.append(pltpu.async_copy(
                batch_hbm.at[pl.ds(o_lo, K)], rows_lo_v.at[l], sem_lo))
            cps.append(pltpu.async_copy(
                batch_hbm.at[pl.ds(o_up, K)], rows_up_v.at[l], sem_up))

        cp_params.wait()
        accs = [jnp.zeros((LANES,), jnp.float32) for _ in range(6)]
        for c in range(hidden // LANES):
            x = plsc.bitcast(aux_v[pl.ds(p_off + c * LANES, LANES)],
                             jnp.float32)
            s = x / (1.0 + jnp.exp(-x))
            row = w2_off + (c * LANES + iota) * 6
            for j in range(6):
                accs[j] = accs[j] + s * f32_gather(aux_v, row + j)
        b2g = f32_gather(aux_v, b2_off + jnp.minimum(iota, 5))
        v = jnp.where(iota < 6, b2g, 0.0)
        for j in range(6):
            v = jnp.where(iota == j, v + jnp.sum(accs[j]), v)
        v_v[...] = v

        for cp in cps:
            cp.wait()

        def fine(rows_v, targets, s_idx, w):
            def g(off):
                return plsc.load_gather(rows_v, [iota, off])
            off = _lower_bound(g, targets, K, fine_steps)
            return jnp.where(s_idx == 0, 0, w + off)

        pos_lo = fine(rows_lo_v, t_lo, s_lo, w_lo)
        pos_up = fine(rows_up_v, t_up, s_up, w_up)
        counts_v[...] = (pos_up - pos_lo).astype(jnp.float32)

        for k in range(6):
            p = k * LANES + iota
            b_local = p // 6
            j = p - 6 * b_local
            cnt = plsc.load_gather(counts_v, [b_local])
            vv = plsc.load_gather(v_v, [j])
            out_v[pl.ds(k * LANES, LANES)] = cnt * vv
        pltpu.sync_copy(out_v, out_hbm.at[pl.ds(wid * LANES * 6, LANES * 6)])

    return body


def kernel(pos, batch, atomic_numbers, W1, b1, W2, b2):
    n = pos.shape[0]
    hidden = b1.shape[0]

    batch_i32 = batch.astype(jnp.int32)
    n_rows = -(-n // K)
    n_sample = -(-(n_rows + 5) // 16) * 16

    p_len = hidden + hidden * 6 + 6
    p_pad = -(-p_len // 16) * 16
    params = jnp.concatenate(
        [b1.astype(jnp.float32),
         jnp.reshape(W2.astype(jnp.float32), (-1,)),
         b2.astype(jnp.float32),
         jnp.zeros((p_pad - p_len,), jnp.float32)])
    aux = jnp.concatenate(
        [batch_i32[::K],
         jnp.full((n_sample - n_rows,), N_STRUCT, jnp.int32),
         lax.bitcast_convert_type(params, jnp.int32)])

    mesh = plsc.VectorSubcoreMesh(core_axis_name="c", subcore_axis_name="s")
    run = functools.partial(
        pl.kernel,
        mesh=mesh,
        compiler_params=pltpu.CompilerParams(needs_layout_passes=False),
        out_type=jax.ShapeDtypeStruct((N_STRUCT * 6,), jnp.float32),
        scratch_types=[
            pltpu.VMEM((n_sample + p_pad,), jnp.int32),
            pltpu.VMEM((LANES, K), jnp.int32),
            pltpu.VMEM((LANES, K), jnp.int32),
            pltpu.VMEM((LANES,), jnp.float32),
            pltpu.VMEM((LANES,), jnp.float32),
            pltpu.VMEM((LANES * 6,), jnp.float32),
            pltpu.SemaphoreType.DMA,
            pltpu.SemaphoreType.DMA,
            pltpu.SemaphoreType.DMA,
        ],
    )(_make_body(n, n_rows, n_sample, hidden))

    forces = jnp.zeros((n, 3), jnp.float32)
    energy = jnp.zeros((N_STRUCT,), jnp.float32)
    stress = run(batch_i32, aux).reshape(N_STRUCT, 6)
    return (forces, energy, stress)
